# baseline (speedup 1.0000x reference)
import functools

import jax
import jax.numpy as jnp
from jax import lax
from jax.experimental import pallas as pl
from jax.experimental.pallas import tpu as pltpu
from jax.experimental.pallas import tpu_sc as plsc

N_NODES = 10000
IN_DIM = 128
HID_DIM = 128
LAT_DIM = 64

NC, NS = 2, 16
NW = NC * NS
K = 128
N_ACC = 10112
CNT_W = 16
RPS = N_ACC // NS


def _mesh():
    return plsc.VectorSubcoreMesh(core_axis_name="c", subcore_axis_name="s",
                                  num_cores=NC, num_subcores=NS)


_SC_PARAMS = pltpu.CompilerParams(use_tc_tiling_on_sc=False)


@functools.lru_cache(maxsize=None)
def _make_count(e_pad, nbuf=8):
    n_chunks = e_pad // NW // K

    @functools.partial(
        pl.kernel,
        out_type=jax.ShapeDtypeStruct((NC, N_ACC, CNT_W), jnp.float32),
        mesh=_mesh(),
        compiler_params=_SC_PARAMS,
        scratch_types=(
            [pltpu.VMEM((n_chunks, K), jnp.int32),
             pltpu.VMEM((K, CNT_W), jnp.float32)]
            + [pltpu.SemaphoreType.DMA] * nbuf
            + [pltpu.VMEM_SHARED((N_ACC, CNT_W), jnp.float32)]
        ),
    )
    def count_kernel(dst_hbm, ones_hbm, zeros_hbm, out_hbm, *refs):
        idxb, onesb = refs[0], refs[1]
        sems = refs[2:2 + nbuf]
        acc = refs[2 + nbuf]
        c = lax.axis_index("c")
        s = lax.axis_index("s")
        wid = c * NS + s
        pltpu.sync_copy(ones_hbm, onesb)
        pltpu.sync_copy(dst_hbm.at[pl.ds(wid * n_chunks, n_chunks)], idxb)
        pltpu.sync_copy(zeros_hbm.at[pl.ds(s * RPS, RPS)],
                        acc.at[pl.ds(s * RPS, RPS)])
        plsc.subcore_barrier()

        @pl.loop(0, n_chunks, step=nbuf)
        def _(j):
            descs = [pltpu.async_copy(onesb, acc.at[idxb.at[j + b]],
                                      sems[b], add=True)
                     for b in range(nbuf)]
            for dsc in descs:
                dsc.wait()

        plsc.subcore_barrier()
        pltpu.sync_copy(acc.at[pl.ds(s * RPS, RPS)],
                        out_hbm.at[c].at[pl.ds(s * RPS, RPS)])

    return count_kernel


@functools.lru_cache(maxsize=None)
def _make_spmm(e_pad, d, nbuf, split=None):
    n_chunks = e_pad // NW // K
    n_groups = n_chunks // nbuf
    nc0, nc1 = split if split else (n_chunks, n_chunks)
    assert nc0 % nbuf == 0 and nc1 % nbuf == 0
    assert NS * (nc0 + nc1) == e_pad // K

    @functools.partial(
        pl.kernel,
        out_type=jax.ShapeDtypeStruct((NC, N_ACC, d), jnp.float32),
        mesh=_mesh(),
        compiler_params=_SC_PARAMS,
        scratch_types=(
            [pltpu.VMEM((2, nbuf, K), jnp.int32),
             pltpu.VMEM((2, nbuf, K), jnp.int32)]
            + [pltpu.VMEM((K, d), jnp.float32)] * nbuf
            + [pltpu.SemaphoreType.DMA] * (1 + 2 * nbuf)
            + [pltpu.VMEM_SHARED((N_ACC, d), jnp.float32)]
        ),
    )
    def spmm_kernel(xs_hbm, src_hbm, dst_hbm, zeros_hbm, out_hbm, *refs):
        srcb, dstb = refs[0], refs[1]
        rows = refs[2:2 + nbuf]
        isem = refs[2 + nbuf]
        gsems = refs[3 + nbuf:3 + 2 * nbuf]
        ssems = refs[3 + 2 * nbuf:3 + 3 * nbuf]
        acc = refs[3 + 3 * nbuf]
        c = lax.axis_index("c")
        s = lax.axis_index("s")
        tbase = jnp.where(c == 0, s * nc0, NS * nc0 + s * nc1)
        my_groups = jnp.where(c == 0, nc0 // nbuf, nc1 // nbuf)
        pltpu.sync_copy(src_hbm.at[pl.ds(tbase, nbuf)], srcb.at[0])
        pltpu.sync_copy(dst_hbm.at[pl.ds(tbase, nbuf)], dstb.at[0])
        pltpu.sync_copy(zeros_hbm.at[pl.ds(s * RPS, RPS)],
                        acc.at[pl.ds(s * RPS, RPS)])
        plsc.subcore_barrier()

        @pl.loop(0, my_groups)
        def _(g):
            p = lax.rem(g, 2)
            pn = 1 - p
            nbase = tbase + (g + 1) * nbuf
            id1 = pltpu.async_copy(src_hbm.at[pl.ds(nbase, nbuf)],
                                   srcb.at[pn], isem)
            id2 = pltpu.async_copy(dst_hbm.at[pl.ds(nbase, nbuf)],
                                   dstb.at[pn], isem)
            gds = [pltpu.async_copy(xs_hbm.at[srcb.at[p, b]], rows[b],
                                    gsems[b])
                   for b in range(nbuf)]
            sds = []
            for b in range(nbuf):
                gds[b].wait()
                sds.append(pltpu.async_copy(rows[b], acc.at[dstb.at[p, b]],
                                            ssems[b], add=True))
            for dsc in sds:
                dsc.wait()
            id1.wait()
            id2.wait()

        plsc.subcore_barrier()
        pltpu.sync_copy(acc.at[pl.ds(s * RPS, RPS)],
                        out_hbm.at[c].at[pl.ds(s * RPS, RPS)])

    return spmm_kernel


def _tc_scale_xw(x, w, cnt0, cnt1):
    def body(x_ref, w_ref, c0_ref, c1_ref, o_ref):
        dinv = lax.rsqrt(c0_ref[...] + c1_ref[...] + 1.0)
        xw = jnp.dot(x_ref[...], w_ref[...], preferred_element_type=jnp.float32)
        o_ref[...] = xw * dinv

    return pl.pallas_call(
        body,
        out_shape=jax.ShapeDtypeStruct((x.shape[0], w.shape[1]), jnp.float32),
    )(x, w, cnt0, cnt1)


def _tc_combine_next(a0, a1, xs, cnt0, cnt1, b, w):
    def body(a0_ref, a1_ref, xs_ref, c0_ref, c1_ref, b_ref, w_ref, o_ref):
        dinv = lax.rsqrt(c0_ref[...] + c1_ref[...] + 1.0)
        h = dinv * (a0_ref[...] + a1_ref[...] + xs_ref[...]) + b_ref[...]
        h = jnp.maximum(h, 0.0)
        hw = jnp.dot(h, w_ref[...], preferred_element_type=jnp.float32)
        o_ref[...] = hw * dinv

    return pl.pallas_call(
        body,
        out_shape=jax.ShapeDtypeStruct((a0.shape[0], w.shape[1]), jnp.float32),
    )(a0, a1, xs, cnt0, cnt1, b, w)


def _tc_final(a0, a1, xs, cnt0, cnt1, b):
    def body(a0_ref, a1_ref, xs_ref, c0_ref, c1_ref, b_ref, o_ref):
        dinv = lax.rsqrt(c0_ref[...] + c1_ref[...] + 1.0)
        o_ref[...] = dinv * (a0_ref[...] + a1_ref[...] + xs_ref[...]) + b_ref[...]

    return pl.pallas_call(
        body,
        out_shape=jax.ShapeDtypeStruct(a0.shape, jnp.float32),
    )(a0, a1, xs, cnt0, cnt1, b)


def kernel(x, edge_index, W1, b1, W2, b2):
    n = x.shape[0]
    e = edge_index.shape[1]
    gran = NW * K * 8
    e_pad = ((e + gran - 1) // gran) * gran

    e_rows = e_pad // K + 8
    src = edge_index[0].astype(jnp.int32)
    dst = edge_index[1].astype(jnp.int32)
    pad = jnp.full((e_rows * K - e,), N_NODES, dtype=jnp.int32)
    src = jnp.concatenate([src, pad]).reshape(e_rows, K)
    dst = jnp.concatenate([dst, pad]).reshape(e_rows, K)

    ones_cnt = jnp.ones((K, CNT_W), jnp.float32)
    zeros_cnt = jnp.zeros((N_ACC, CNT_W), jnp.float32)
    zeros_h = jnp.zeros((N_ACC, HID_DIM), jnp.float32)
    zeros_l = jnp.zeros((N_ACC, LAT_DIM), jnp.float32)

    counts = _make_count(e_pad)(dst, ones_cnt, zeros_cnt)
    cnt0 = counts[0, :n, :1]
    cnt1 = counts[1, :n, :1]

    xs1 = _tc_scale_xw(x, W1, cnt0, cnt1)
    xs1_pad = jnp.pad(xs1, ((0, N_ACC - n), (0, 0)))

    acc1 = _make_spmm(e_pad, HID_DIM, 2, (160, 0))(xs1_pad, src, dst, zeros_h)
    xs2 = _tc_combine_next(acc1[0, :n], acc1[1, :n], xs1, cnt0, cnt1,
                           b1.reshape(1, -1), W2)
    xs2_pad = jnp.pad(xs2, ((0, N_ACC - n), (0, 0)))

    acc2 = _make_spmm(e_pad, LAT_DIM, 8, (160, 0))(xs2_pad, src, dst, zeros_l)
    out = _tc_final(acc2[0, :n], acc2[1, :n], xs2, cnt0, cnt1,
                    b2.reshape(1, -1))
    return out


# speedup vs baseline: 2.1890x; 2.1890x over previous
import functools

import jax
import jax.numpy as jnp
from jax import lax
from jax.experimental import pallas as pl
from jax.experimental.pallas import tpu as pltpu
from jax.experimental.pallas import tpu_sc as plsc

N_NODES = 10000
IN_DIM = 128
HID_DIM = 128
LAT_DIM = 64

NC, NS = 2, 16
NW = NC * NS
K = 128
N_ACC = 10112
CNT_W = 16
RPS = N_ACC // NS


def _mesh():
    return plsc.VectorSubcoreMesh(core_axis_name="c", subcore_axis_name="s",
                                  num_cores=NC, num_subcores=NS)


_SC_PARAMS = pltpu.CompilerParams(use_tc_tiling_on_sc=False)


@functools.lru_cache(maxsize=None)
def _make_count(e_pad, nbuf=8):
    n_chunks = e_pad // NW // K

    @functools.partial(
        pl.kernel,
        out_type=jax.ShapeDtypeStruct((NC, N_ACC, CNT_W), jnp.float32),
        mesh=_mesh(),
        compiler_params=_SC_PARAMS,
        scratch_types=(
            [pltpu.VMEM((n_chunks, K), jnp.int32),
             pltpu.VMEM((K, CNT_W), jnp.float32)]
            + [pltpu.SemaphoreType.DMA] * nbuf
            + [pltpu.VMEM_SHARED((N_ACC, CNT_W), jnp.float32)]
        ),
    )
    def count_kernel(dst_hbm, ones_hbm, zeros_hbm, out_hbm, *refs):
        idxb, onesb = refs[0], refs[1]
        sems = refs[2:2 + nbuf]
        acc = refs[2 + nbuf]
        c = lax.axis_index("c")
        s = lax.axis_index("s")
        wid = c * NS + s
        pltpu.sync_copy(ones_hbm, onesb)
        pltpu.sync_copy(dst_hbm.at[pl.ds(wid * n_chunks, n_chunks)], idxb)
        pltpu.sync_copy(zeros_hbm.at[pl.ds(s * RPS, RPS)],
                        acc.at[pl.ds(s * RPS, RPS)])
        plsc.subcore_barrier()

        @pl.loop(0, n_chunks, step=nbuf)
        def _(j):
            descs = [pltpu.async_copy(onesb, acc.at[idxb.at[j + b]],
                                      sems[b], add=True)
                     for b in range(nbuf)]
            for dsc in descs:
                dsc.wait()

        plsc.subcore_barrier()
        pltpu.sync_copy(acc.at[pl.ds(s * RPS, RPS)],
                        out_hbm.at[c].at[pl.ds(s * RPS, RPS)])

    return count_kernel


@functools.lru_cache(maxsize=None)
def _make_spmm(e_pad, d, nbuf, split=None):
    n_chunks = e_pad // NW // K
    n_groups = n_chunks // nbuf
    nc0, nc1 = split if split else (n_chunks, n_chunks)
    assert nc0 % nbuf == 0 and nc1 % nbuf == 0
    assert NS * (nc0 + nc1) == e_pad // K

    @functools.partial(
        pl.kernel,
        out_type=jax.ShapeDtypeStruct((NC, N_ACC, d), jnp.float32),
        mesh=_mesh(),
        compiler_params=_SC_PARAMS,
        scratch_types=(
            [pltpu.VMEM((2, nbuf, K), jnp.int32),
             pltpu.VMEM((2, nbuf, K), jnp.int32)]
            + [pltpu.VMEM((K, d), jnp.float32)] * nbuf
            + [pltpu.SemaphoreType.DMA] * (1 + 2 * nbuf)
            + [pltpu.VMEM_SHARED((N_ACC, d), jnp.float32)]
        ),
    )
    def spmm_kernel(xs_hbm, src_hbm, dst_hbm, zeros_hbm, out_hbm, *refs):
        srcb, dstb = refs[0], refs[1]
        rows = refs[2:2 + nbuf]
        isem = refs[2 + nbuf]
        gsems = refs[3 + nbuf:3 + 2 * nbuf]
        ssems = refs[3 + 2 * nbuf:3 + 3 * nbuf]
        acc = refs[3 + 3 * nbuf]
        c = lax.axis_index("c")
        s = lax.axis_index("s")
        tbase = jnp.where(c == 0, s * nc0, NS * nc0 + s * nc1)
        my_groups = jnp.where(c == 0, nc0 // nbuf, nc1 // nbuf)
        pltpu.sync_copy(src_hbm.at[pl.ds(tbase, nbuf)], srcb.at[0])
        pltpu.sync_copy(dst_hbm.at[pl.ds(tbase, nbuf)], dstb.at[0])
        pltpu.sync_copy(zeros_hbm.at[pl.ds(s * RPS, RPS)],
                        acc.at[pl.ds(s * RPS, RPS)])
        plsc.subcore_barrier()

        @pl.loop(0, my_groups)
        def _(g):
            p = lax.rem(g, 2)
            pn = 1 - p
            nbase = tbase + (g + 1) * nbuf
            id1 = pltpu.async_copy(src_hbm.at[pl.ds(nbase, nbuf)],
                                   srcb.at[pn], isem)
            id2 = pltpu.async_copy(dst_hbm.at[pl.ds(nbase, nbuf)],
                                   dstb.at[pn], isem)
            gds = [pltpu.async_copy(xs_hbm.at[srcb.at[p, b]], rows[b],
                                    gsems[b])
                   for b in range(nbuf)]
            sds = []
            for b in range(nbuf):
                gds[b].wait()
                sds.append(pltpu.async_copy(rows[b], acc.at[dstb.at[p, b]],
                                            ssems[b], add=True))
            for dsc in sds:
                dsc.wait()
            id1.wait()
            id2.wait()

        plsc.subcore_barrier()
        pltpu.sync_copy(acc.at[pl.ds(s * RPS, RPS)],
                        out_hbm.at[c].at[pl.ds(s * RPS, RPS)])

    return spmm_kernel


@functools.lru_cache(maxsize=None)
def _make_spmm_spmem(e_pad, d, nbuf, split=None):
    n_chunks = e_pad // NW // K
    nc0, nc1 = split if split else (n_chunks, n_chunks)
    assert nc0 % nbuf == 0 and nc1 % nbuf == 0
    assert NS * (nc0 + nc1) == e_pad // K

    @functools.partial(
        pl.kernel,
        out_type=jax.ShapeDtypeStruct((NC, N_ACC, d), jnp.float32),
        mesh=_mesh(),
        compiler_params=_SC_PARAMS,
        scratch_types=(
            [pltpu.VMEM((2, nbuf, K), jnp.int32),
             pltpu.VMEM((2, nbuf, K), jnp.int32)]
            + [pltpu.VMEM((K, d), jnp.float32)] * nbuf
            + [pltpu.SemaphoreType.DMA] * (1 + 2 * nbuf)
            + [pltpu.VMEM_SHARED((N_ACC, d), jnp.float32),
               pltpu.VMEM_SHARED((N_ACC, d), jnp.float32)]
        ),
    )
    def spmm_kernel(xs_hbm, src_hbm, dst_hbm, zeros_hbm, out_hbm, *refs):
        srcb, dstb = refs[0], refs[1]
        rows = refs[2:2 + nbuf]
        isem = refs[2 + nbuf]
        gsems = refs[3 + nbuf:3 + 2 * nbuf]
        ssems = refs[3 + 2 * nbuf:3 + 3 * nbuf]
        xs_sp = refs[3 + 3 * nbuf]
        acc = refs[4 + 3 * nbuf]
        c = lax.axis_index("c")
        s = lax.axis_index("s")
        tbase = jnp.where(c == 0, s * nc0, NS * nc0 + s * nc1)
        my_groups = jnp.where(c == 0, nc0 // nbuf, nc1 // nbuf)
        pltpu.sync_copy(src_hbm.at[pl.ds(tbase, nbuf)], srcb.at[0])
        pltpu.sync_copy(dst_hbm.at[pl.ds(tbase, nbuf)], dstb.at[0])
        pltpu.sync_copy(xs_hbm.at[pl.ds(s * RPS, RPS)],
                        xs_sp.at[pl.ds(s * RPS, RPS)])
        pltpu.sync_copy(zeros_hbm.at[pl.ds(s * RPS, RPS)],
                        acc.at[pl.ds(s * RPS, RPS)])
        plsc.subcore_barrier()

        @pl.loop(0, my_groups)
        def _(g):
            p = lax.rem(g, 2)
            pn = 1 - p
            nbase = tbase + (g + 1) * nbuf
            id1 = pltpu.async_copy(src_hbm.at[pl.ds(nbase, nbuf)],
                                   srcb.at[pn], isem)
            id2 = pltpu.async_copy(dst_hbm.at[pl.ds(nbase, nbuf)],
                                   dstb.at[pn], isem)
            gds = [pltpu.async_copy(xs_sp.at[srcb.at[p, b]], rows[b],
                                    gsems[b])
                   for b in range(nbuf)]
            sds = []
            for b in range(nbuf):
                gds[b].wait()
                sds.append(pltpu.async_copy(rows[b], acc.at[dstb.at[p, b]],
                                            ssems[b], add=True))
            for dsc in sds:
                dsc.wait()
            id1.wait()
            id2.wait()

        plsc.subcore_barrier()
        pltpu.sync_copy(acc.at[pl.ds(s * RPS, RPS)],
                        out_hbm.at[c].at[pl.ds(s * RPS, RPS)])

    return spmm_kernel


def _tc_scale_xw(x, w, cnt0, cnt1):
    def body(x_ref, w_ref, c0_ref, c1_ref, o_ref):
        dinv = lax.rsqrt(c0_ref[...] + c1_ref[...] + 1.0)
        xw = jnp.dot(x_ref[...], w_ref[...], preferred_element_type=jnp.float32)
        o_ref[...] = xw * dinv

    return pl.pallas_call(
        body,
        out_shape=jax.ShapeDtypeStruct((x.shape[0], w.shape[1]), jnp.float32),
    )(x, w, cnt0, cnt1)


def _tc_combine_next(a0l, a1l, a0r, a1r, xs, cnt0, cnt1, b, w):
    half = xs.shape[1] // 2

    def body(a0l_ref, a1l_ref, a0r_ref, a1r_ref, xs_ref, c0_ref, c1_ref,
             b_ref, w_ref, o_ref):
        dinv = lax.rsqrt(c0_ref[...] + c1_ref[...] + 1.0)
        left = a0l_ref[...] + a1l_ref[...] + xs_ref[:, :half]
        right = a0r_ref[...] + a1r_ref[...] + xs_ref[:, half:]
        h = dinv * jnp.concatenate([left, right], axis=1) + b_ref[...]
        h = jnp.maximum(h, 0.0)
        hw = jnp.dot(h, w_ref[...], preferred_element_type=jnp.float32)
        o_ref[...] = hw * dinv

    return pl.pallas_call(
        body,
        out_shape=jax.ShapeDtypeStruct((xs.shape[0], w.shape[1]), jnp.float32),
    )(a0l, a1l, a0r, a1r, xs, cnt0, cnt1, b, w)


def _tc_final(a0, a1, xs, cnt0, cnt1, b):
    def body(a0_ref, a1_ref, xs_ref, c0_ref, c1_ref, b_ref, o_ref):
        dinv = lax.rsqrt(c0_ref[...] + c1_ref[...] + 1.0)
        o_ref[...] = dinv * (a0_ref[...] + a1_ref[...] + xs_ref[...]) + b_ref[...]

    return pl.pallas_call(
        body,
        out_shape=jax.ShapeDtypeStruct(a0.shape, jnp.float32),
    )(a0, a1, xs, cnt0, cnt1, b)


def kernel(x, edge_index, W1, b1, W2, b2):
    n = x.shape[0]
    e = edge_index.shape[1]
    gran = NW * K * 8
    e_pad = ((e + gran - 1) // gran) * gran

    e_rows = e_pad // K + 8
    src = edge_index[0].astype(jnp.int32)
    dst = edge_index[1].astype(jnp.int32)
    pad = jnp.full((e_rows * K - e,), N_NODES, dtype=jnp.int32)
    src = jnp.concatenate([src, pad]).reshape(e_rows, K)
    dst = jnp.concatenate([dst, pad]).reshape(e_rows, K)

    ones_cnt = jnp.ones((K, CNT_W), jnp.float32)
    zeros_cnt = jnp.zeros((N_ACC, CNT_W), jnp.float32)
    zeros_h = jnp.zeros((N_ACC, HID_DIM), jnp.float32)
    zeros_l = jnp.zeros((N_ACC, LAT_DIM), jnp.float32)

    counts = _make_count(e_pad)(dst, ones_cnt, zeros_cnt)
    cnt0 = counts[0, :n, :1]
    cnt1 = counts[1, :n, :1]

    xs1 = _tc_scale_xw(x, W1, cnt0, cnt1)
    xs1_pad = jnp.pad(xs1, ((0, N_ACC - n), (0, 0)))
    half = HID_DIM // 2

    spmm64 = _make_spmm_spmem(e_pad, half, 4)
    acc1l = spmm64(xs1_pad[:, :half], src, dst, zeros_l)
    acc1r = spmm64(xs1_pad[:, half:], src, dst, zeros_l)
    xs2 = _tc_combine_next(acc1l[0, :n], acc1l[1, :n],
                           acc1r[0, :n], acc1r[1, :n], xs1, cnt0, cnt1,
                           b1.reshape(1, -1), W2)
    xs2_pad = jnp.pad(xs2, ((0, N_ACC - n), (0, 0)))

    acc2 = spmm64(xs2_pad, src, dst, zeros_l)
    out = _tc_final(acc2[0, :n], acc2[1, :n], xs2, cnt0, cnt1,
                    b2.reshape(1, -1))
    return out


# speedup vs baseline: 2.3377x; 1.0679x over previous
import functools

import jax
import jax.numpy as jnp
from jax import lax
from jax.experimental import pallas as pl
from jax.experimental.pallas import tpu as pltpu
from jax.experimental.pallas import tpu_sc as plsc

N_NODES = 10000
IN_DIM = 128
HID_DIM = 128
LAT_DIM = 64

NC, NS = 2, 16
NW = NC * NS
K = 128
N_ACC = 10112
CNT_W = 16
RPS = N_ACC // NS


def _mesh():
    return plsc.VectorSubcoreMesh(core_axis_name="c", subcore_axis_name="s",
                                  num_cores=NC, num_subcores=NS)


_SC_PARAMS = pltpu.CompilerParams(use_tc_tiling_on_sc=False)


@functools.lru_cache(maxsize=None)
def _make_count(e_pad, nbuf=8):
    n_chunks = e_pad // NW // K

    @functools.partial(
        pl.kernel,
        out_type=jax.ShapeDtypeStruct((NC, N_ACC, CNT_W), jnp.float32),
        mesh=_mesh(),
        compiler_params=_SC_PARAMS,
        scratch_types=(
            [pltpu.VMEM((n_chunks, K), jnp.int32),
             pltpu.VMEM((K, CNT_W), jnp.float32)]
            + [pltpu.SemaphoreType.DMA] * nbuf
            + [pltpu.VMEM_SHARED((N_ACC, CNT_W), jnp.float32)]
        ),
    )
    def count_kernel(dst_hbm, ones_hbm, zeros_hbm, out_hbm, *refs):
        idxb, onesb = refs[0], refs[1]
        sems = refs[2:2 + nbuf]
        acc = refs[2 + nbuf]
        c = lax.axis_index("c")
        s = lax.axis_index("s")
        wid = c * NS + s
        pltpu.sync_copy(ones_hbm, onesb)
        pltpu.sync_copy(dst_hbm.at[pl.ds(wid * n_chunks, n_chunks)], idxb)
        pltpu.sync_copy(zeros_hbm.at[pl.ds(s * RPS, RPS)],
                        acc.at[pl.ds(s * RPS, RPS)])
        plsc.subcore_barrier()

        @pl.loop(0, n_chunks, step=nbuf)
        def _(j):
            descs = [pltpu.async_copy(onesb, acc.at[idxb.at[j + b]],
                                      sems[b], add=True)
                     for b in range(nbuf)]
            for dsc in descs:
                dsc.wait()

        plsc.subcore_barrier()
        pltpu.sync_copy(acc.at[pl.ds(s * RPS, RPS)],
                        out_hbm.at[c].at[pl.ds(s * RPS, RPS)])

    return count_kernel


@functools.lru_cache(maxsize=None)
def _make_spmm(e_pad, d, nbuf, split=None):
    n_chunks = e_pad // NW // K
    n_groups = n_chunks // nbuf
    nc0, nc1 = split if split else (n_chunks, n_chunks)
    assert nc0 % nbuf == 0 and nc1 % nbuf == 0
    assert NS * (nc0 + nc1) == e_pad // K

    @functools.partial(
        pl.kernel,
        out_type=jax.ShapeDtypeStruct((NC, N_ACC, d), jnp.float32),
        mesh=_mesh(),
        compiler_params=_SC_PARAMS,
        scratch_types=(
            [pltpu.VMEM((2, nbuf, K), jnp.int32),
             pltpu.VMEM((2, nbuf, K), jnp.int32)]
            + [pltpu.VMEM((K, d), jnp.float32)] * nbuf
            + [pltpu.SemaphoreType.DMA] * (1 + 2 * nbuf)
            + [pltpu.VMEM_SHARED((N_ACC, d), jnp.float32)]
        ),
    )
    def spmm_kernel(xs_hbm, src_hbm, dst_hbm, zeros_hbm, out_hbm, *refs):
        srcb, dstb = refs[0], refs[1]
        rows = refs[2:2 + nbuf]
        isem = refs[2 + nbuf]
        gsems = refs[3 + nbuf:3 + 2 * nbuf]
        ssems = refs[3 + 2 * nbuf:3 + 3 * nbuf]
        acc = refs[3 + 3 * nbuf]
        c = lax.axis_index("c")
        s = lax.axis_index("s")
        tbase = jnp.where(c == 0, s * nc0, NS * nc0 + s * nc1)
        my_groups = jnp.where(c == 0, nc0 // nbuf, nc1 // nbuf)
        pltpu.sync_copy(src_hbm.at[pl.ds(tbase, nbuf)], srcb.at[0])
        pltpu.sync_copy(dst_hbm.at[pl.ds(tbase, nbuf)], dstb.at[0])
        pltpu.sync_copy(zeros_hbm.at[pl.ds(s * RPS, RPS)],
                        acc.at[pl.ds(s * RPS, RPS)])
        plsc.subcore_barrier()

        @pl.loop(0, my_groups)
        def _(g):
            p = lax.rem(g, 2)
            pn = 1 - p
            nbase = tbase + (g + 1) * nbuf
            id1 = pltpu.async_copy(src_hbm.at[pl.ds(nbase, nbuf)],
                                   srcb.at[pn], isem)
            id2 = pltpu.async_copy(dst_hbm.at[pl.ds(nbase, nbuf)],
                                   dstb.at[pn], isem)
            gds = [pltpu.async_copy(xs_hbm.at[srcb.at[p, b]], rows[b],
                                    gsems[b])
                   for b in range(nbuf)]
            sds = []
            for b in range(nbuf):
                gds[b].wait()
                sds.append(pltpu.async_copy(rows[b], acc.at[dstb.at[p, b]],
                                            ssems[b], add=True))
            for dsc in sds:
                dsc.wait()
            id1.wait()
            id2.wait()

        plsc.subcore_barrier()
        pltpu.sync_copy(acc.at[pl.ds(s * RPS, RPS)],
                        out_hbm.at[c].at[pl.ds(s * RPS, RPS)])

    return spmm_kernel


@functools.lru_cache(maxsize=None)
def _make_spmm_spmem(e_pad, d, nbuf, split=None):
    n_chunks = e_pad // NW // K
    nc0, nc1 = split if split else (n_chunks, n_chunks)
    assert nc0 % nbuf == 0 and nc1 % nbuf == 0
    assert NS * (nc0 + nc1) == e_pad // K

    @functools.partial(
        pl.kernel,
        out_type=jax.ShapeDtypeStruct((NC, N_ACC, d), jnp.float32),
        mesh=_mesh(),
        compiler_params=_SC_PARAMS,
        scratch_types=(
            [pltpu.VMEM((2, nbuf, K), jnp.int32),
             pltpu.VMEM((2, nbuf, K), jnp.int32)]
            + [pltpu.VMEM((K, d), jnp.float32)] * nbuf
            + [pltpu.SemaphoreType.DMA] * (1 + 2 * nbuf)
            + [pltpu.VMEM_SHARED((N_ACC, d), jnp.float32),
               pltpu.VMEM_SHARED((N_ACC, d), jnp.float32)]
        ),
    )
    def spmm_kernel(xs_hbm, src_hbm, dst_hbm, zeros_hbm, out_hbm, *refs):
        srcb, dstb = refs[0], refs[1]
        rows = refs[2:2 + nbuf]
        isem = refs[2 + nbuf]
        gsems = refs[3 + nbuf:3 + 2 * nbuf]
        ssems = refs[3 + 2 * nbuf:3 + 3 * nbuf]
        xs_sp = refs[3 + 3 * nbuf]
        acc = refs[4 + 3 * nbuf]
        c = lax.axis_index("c")
        s = lax.axis_index("s")
        tbase = jnp.where(c == 0, s * nc0, NS * nc0 + s * nc1)
        my_groups = jnp.where(c == 0, nc0 // nbuf, nc1 // nbuf)
        pltpu.sync_copy(src_hbm.at[pl.ds(tbase, nbuf)], srcb.at[0])
        pltpu.sync_copy(dst_hbm.at[pl.ds(tbase, nbuf)], dstb.at[0])
        pltpu.sync_copy(xs_hbm.at[pl.ds(s * RPS, RPS)],
                        xs_sp.at[pl.ds(s * RPS, RPS)])
        pltpu.sync_copy(zeros_hbm.at[pl.ds(s * RPS, RPS)],
                        acc.at[pl.ds(s * RPS, RPS)])
        plsc.subcore_barrier()

        @pl.loop(0, my_groups)
        def _(g):
            p = lax.rem(g, 2)
            pn = 1 - p
            nbase = tbase + (g + 1) * nbuf
            id1 = pltpu.async_copy(src_hbm.at[pl.ds(nbase, nbuf)],
                                   srcb.at[pn], isem)
            id2 = pltpu.async_copy(dst_hbm.at[pl.ds(nbase, nbuf)],
                                   dstb.at[pn], isem)
            gds = [pltpu.async_copy(xs_sp.at[srcb.at[p, b]], rows[b],
                                    gsems[b])
                   for b in range(nbuf)]
            sds = []
            for b in range(nbuf):
                gds[b].wait()
                sds.append(pltpu.async_copy(rows[b], acc.at[dstb.at[p, b]],
                                            ssems[b], add=True))
            for dsc in sds:
                dsc.wait()
            id1.wait()
            id2.wait()

        plsc.subcore_barrier()
        pltpu.sync_copy(acc.at[pl.ds(s * RPS, RPS)],
                        out_hbm.at[c].at[pl.ds(s * RPS, RPS)])

    return spmm_kernel


def _tc_scale_xw(x, w, cnt0, cnt1):
    def body(x_ref, w_ref, c0_ref, c1_ref, o_ref):
        dinv = lax.rsqrt(c0_ref[...] + c1_ref[...] + 1.0)
        xw = jnp.dot(x_ref[...], w_ref[...], preferred_element_type=jnp.float32)
        o_ref[...] = xw * dinv

    return pl.pallas_call(
        body,
        out_shape=jax.ShapeDtypeStruct((x.shape[0], w.shape[1]), jnp.float32),
    )(x, w, cnt0, cnt1)


def _tc_combine_next(a0l, a1l, a0r, a1r, xs, cnt0, cnt1, b, w):
    half = xs.shape[1] // 2

    def body(a0l_ref, a1l_ref, a0r_ref, a1r_ref, xs_ref, c0_ref, c1_ref,
             b_ref, w_ref, o_ref):
        dinv = lax.rsqrt(c0_ref[...] + c1_ref[...] + 1.0)
        left = a0l_ref[...] + a1l_ref[...] + xs_ref[:, :half]
        right = a0r_ref[...] + a1r_ref[...] + xs_ref[:, half:]
        h = dinv * jnp.concatenate([left, right], axis=1) + b_ref[...]
        h = jnp.maximum(h, 0.0)
        hw = jnp.dot(h, w_ref[...], preferred_element_type=jnp.float32)
        o_ref[...] = hw * dinv

    return pl.pallas_call(
        body,
        out_shape=jax.ShapeDtypeStruct((xs.shape[0], w.shape[1]), jnp.float32),
    )(a0l, a1l, a0r, a1r, xs, cnt0, cnt1, b, w)


def _tc_final(a0, a1, xs, cnt0, cnt1, b):
    def body(a0_ref, a1_ref, xs_ref, c0_ref, c1_ref, b_ref, o_ref):
        dinv = lax.rsqrt(c0_ref[...] + c1_ref[...] + 1.0)
        o_ref[...] = dinv * (a0_ref[...] + a1_ref[...] + xs_ref[...]) + b_ref[...]

    return pl.pallas_call(
        body,
        out_shape=jax.ShapeDtypeStruct(a0.shape, jnp.float32),
    )(a0, a1, xs, cnt0, cnt1, b)


def kernel(x, edge_index, W1, b1, W2, b2):
    n = x.shape[0]
    e = edge_index.shape[1]
    gran = NW * K * 8
    e_pad = ((e + gran - 1) // gran) * gran

    e_rows = e_pad // K + 8
    src = edge_index[0].astype(jnp.int32)
    dst = edge_index[1].astype(jnp.int32)
    pad = jnp.full((e_rows * K - e,), N_NODES, dtype=jnp.int32)
    src = jnp.concatenate([src, pad]).reshape(e_rows, K)
    dst = jnp.concatenate([dst, pad]).reshape(e_rows, K)

    ones_cnt = jnp.ones((K, CNT_W), jnp.float32)
    zeros_cnt = jnp.zeros((N_ACC, CNT_W), jnp.float32)
    zeros_h = jnp.zeros((N_ACC, HID_DIM), jnp.float32)
    zeros_l = jnp.zeros((N_ACC, LAT_DIM), jnp.float32)

    counts = _make_count(e_pad)(dst, ones_cnt, zeros_cnt)
    cnt0 = counts[0, :n, :1]
    cnt1 = counts[1, :n, :1]

    xs1 = _tc_scale_xw(x, W1, cnt0, cnt1)
    xs1_pad = jnp.pad(xs1, ((0, N_ACC - n), (0, 0)))
    half = HID_DIM // 2

    spmm64 = _make_spmm_spmem(e_pad, half, 4, (92, 68))
    acc1l = spmm64(xs1_pad[:, :half], src, dst, zeros_l)
    acc1r = spmm64(xs1_pad[:, half:], src, dst, zeros_l)
    xs2 = _tc_combine_next(acc1l[0, :n], acc1l[1, :n],
                           acc1r[0, :n], acc1r[1, :n], xs1, cnt0, cnt1,
                           b1.reshape(1, -1), W2)
    xs2_pad = jnp.pad(xs2, ((0, N_ACC - n), (0, 0)))

    acc2 = spmm64(xs2_pad, src, dst, zeros_l)
    out = _tc_final(acc2[0, :n], acc2[1, :n], xs2, cnt0, cnt1,
                    b2.reshape(1, -1))
    return out


# speedup vs baseline: 2.5349x; 1.0844x over previous
import functools

import jax
import jax.numpy as jnp
from jax import lax
from jax.experimental import pallas as pl
from jax.experimental.pallas import tpu as pltpu
from jax.experimental.pallas import tpu_sc as plsc

N_NODES = 10000
IN_DIM = 128
HID_DIM = 128
LAT_DIM = 64

NC, NS = 2, 16
NW = NC * NS
K = 128
N_ACC = 10112
CNT_W = 16
RPS = N_ACC // NS


def _mesh():
    return plsc.VectorSubcoreMesh(core_axis_name="c", subcore_axis_name="s",
                                  num_cores=NC, num_subcores=NS)


_SC_PARAMS = pltpu.CompilerParams(use_tc_tiling_on_sc=False)


@functools.lru_cache(maxsize=None)
def _make_count(e_pad, nbuf=8):
    n_chunks = e_pad // NW // K

    @functools.partial(
        pl.kernel,
        out_type=jax.ShapeDtypeStruct((NC, N_ACC, CNT_W), jnp.float32),
        mesh=_mesh(),
        compiler_params=_SC_PARAMS,
        scratch_types=(
            [pltpu.VMEM((n_chunks, K), jnp.int32),
             pltpu.VMEM((K, CNT_W), jnp.float32)]
            + [pltpu.SemaphoreType.DMA] * nbuf
            + [pltpu.VMEM_SHARED((N_ACC, CNT_W), jnp.float32)]
        ),
    )
    def count_kernel(dst_hbm, ones_hbm, zeros_hbm, out_hbm, *refs):
        idxb, onesb = refs[0], refs[1]
        sems = refs[2:2 + nbuf]
        acc = refs[2 + nbuf]
        c = lax.axis_index("c")
        s = lax.axis_index("s")
        wid = c * NS + s
        pltpu.sync_copy(ones_hbm, onesb)
        pltpu.sync_copy(dst_hbm.at[pl.ds(wid * n_chunks, n_chunks)], idxb)
        pltpu.sync_copy(zeros_hbm.at[pl.ds(s * RPS, RPS)],
                        acc.at[pl.ds(s * RPS, RPS)])
        plsc.subcore_barrier()

        @pl.loop(0, n_chunks, step=nbuf)
        def _(j):
            descs = [pltpu.async_copy(onesb, acc.at[idxb.at[j + b]],
                                      sems[b], add=True)
                     for b in range(nbuf)]
            for dsc in descs:
                dsc.wait()

        plsc.subcore_barrier()
        pltpu.sync_copy(acc.at[pl.ds(s * RPS, RPS)],
                        out_hbm.at[c].at[pl.ds(s * RPS, RPS)])

    return count_kernel


@functools.lru_cache(maxsize=None)
def _make_spmm(e_pad, d, nbuf, split=None):
    n_chunks = e_pad // NW // K
    n_groups = n_chunks // nbuf
    nc0, nc1 = split if split else (n_chunks, n_chunks)
    assert nc0 % nbuf == 0 and nc1 % nbuf == 0
    assert NS * (nc0 + nc1) == e_pad // K

    @functools.partial(
        pl.kernel,
        out_type=jax.ShapeDtypeStruct((NC, N_ACC, d), jnp.float32),
        mesh=_mesh(),
        compiler_params=_SC_PARAMS,
        scratch_types=(
            [pltpu.VMEM((2, nbuf, K), jnp.int32),
             pltpu.VMEM((2, nbuf, K), jnp.int32)]
            + [pltpu.VMEM((K, d), jnp.float32)] * nbuf
            + [pltpu.SemaphoreType.DMA] * (1 + 2 * nbuf)
            + [pltpu.VMEM_SHARED((N_ACC, d), jnp.float32)]
        ),
    )
    def spmm_kernel(xs_hbm, src_hbm, dst_hbm, zeros_hbm, out_hbm, *refs):
        srcb, dstb = refs[0], refs[1]
        rows = refs[2:2 + nbuf]
        isem = refs[2 + nbuf]
        gsems = refs[3 + nbuf:3 + 2 * nbuf]
        ssems = refs[3 + 2 * nbuf:3 + 3 * nbuf]
        acc = refs[3 + 3 * nbuf]
        c = lax.axis_index("c")
        s = lax.axis_index("s")
        tbase = jnp.where(c == 0, s * nc0, NS * nc0 + s * nc1)
        my_groups = jnp.where(c == 0, nc0 // nbuf, nc1 // nbuf)
        pltpu.sync_copy(src_hbm.at[pl.ds(tbase, nbuf)], srcb.at[0])
        pltpu.sync_copy(dst_hbm.at[pl.ds(tbase, nbuf)], dstb.at[0])
        pltpu.sync_copy(zeros_hbm.at[pl.ds(s * RPS, RPS)],
                        acc.at[pl.ds(s * RPS, RPS)])
        plsc.subcore_barrier()

        @pl.loop(0, my_groups)
        def _(g):
            p = lax.rem(g, 2)
            pn = 1 - p
            nbase = tbase + (g + 1) * nbuf
            id1 = pltpu.async_copy(src_hbm.at[pl.ds(nbase, nbuf)],
                                   srcb.at[pn], isem)
            id2 = pltpu.async_copy(dst_hbm.at[pl.ds(nbase, nbuf)],
                                   dstb.at[pn], isem)
            gds = [pltpu.async_copy(xs_hbm.at[srcb.at[p, b]], rows[b],
                                    gsems[b])
                   for b in range(nbuf)]
            sds = []
            for b in range(nbuf):
                gds[b].wait()
                sds.append(pltpu.async_copy(rows[b], acc.at[dstb.at[p, b]],
                                            ssems[b], add=True))
            for dsc in sds:
                dsc.wait()
            id1.wait()
            id2.wait()

        plsc.subcore_barrier()
        pltpu.sync_copy(acc.at[pl.ds(s * RPS, RPS)],
                        out_hbm.at[c].at[pl.ds(s * RPS, RPS)])

    return spmm_kernel


@functools.lru_cache(maxsize=None)
def _make_spmm_spmem(e_pad, d, nbuf, split=None, col=0):
    n_chunks = e_pad // NW // K
    nc0, nc1 = split if split else (n_chunks, n_chunks)
    assert nc0 % nbuf == 0 and nc1 % nbuf == 0
    assert NS * (nc0 + nc1) == e_pad // K

    @functools.partial(
        pl.kernel,
        out_type=jax.ShapeDtypeStruct((NC, N_ACC, d), jnp.float32),
        mesh=_mesh(),
        compiler_params=_SC_PARAMS,
        scratch_types=(
            [pltpu.VMEM((2, nbuf, K), jnp.int32),
             pltpu.VMEM((2, nbuf, K), jnp.int32)]
            + [pltpu.VMEM((K, d), jnp.float32)] * nbuf
            + [pltpu.SemaphoreType.DMA] * (1 + 2 * nbuf)
            + [pltpu.VMEM_SHARED((N_ACC, d), jnp.float32),
               pltpu.VMEM_SHARED((N_ACC, d), jnp.float32)]
        ),
    )
    def spmm_kernel(xs_hbm, src_hbm, dst_hbm, zeros_hbm, out_hbm, *refs):
        srcb, dstb = refs[0], refs[1]
        rows = refs[2:2 + nbuf]
        isem = refs[2 + nbuf]
        gsems = refs[3 + nbuf:3 + 2 * nbuf]
        ssems = refs[3 + 2 * nbuf:3 + 3 * nbuf]
        xs_sp = refs[3 + 3 * nbuf]
        acc = refs[4 + 3 * nbuf]
        c = lax.axis_index("c")
        s = lax.axis_index("s")
        tbase = jnp.where(c == 0, s * nc0, NS * nc0 + s * nc1)
        my_groups = jnp.where(c == 0, nc0 // nbuf, nc1 // nbuf)
        pltpu.sync_copy(src_hbm.at[pl.ds(tbase, nbuf)], srcb.at[0])
        pltpu.sync_copy(dst_hbm.at[pl.ds(tbase, nbuf)], dstb.at[0])
        pltpu.sync_copy(xs_hbm.at[pl.ds(s * RPS, RPS), pl.ds(col, d)],
                        xs_sp.at[pl.ds(s * RPS, RPS)])
        pltpu.sync_copy(zeros_hbm.at[pl.ds(s * RPS, RPS)],
                        acc.at[pl.ds(s * RPS, RPS)])
        plsc.subcore_barrier()

        @pl.loop(0, my_groups)
        def _(g):
            p = lax.rem(g, 2)
            pn = 1 - p
            nbase = tbase + (g + 1) * nbuf
            id1 = pltpu.async_copy(src_hbm.at[pl.ds(nbase, nbuf)],
                                   srcb.at[pn], isem)
            id2 = pltpu.async_copy(dst_hbm.at[pl.ds(nbase, nbuf)],
                                   dstb.at[pn], isem)
            gds = [pltpu.async_copy(xs_sp.at[srcb.at[p, b]], rows[b],
                                    gsems[b])
                   for b in range(nbuf)]
            sds = []
            for b in range(nbuf):
                gds[b].wait()
                sds.append(pltpu.async_copy(rows[b], acc.at[dstb.at[p, b]],
                                            ssems[b], add=True))
            for dsc in sds:
                dsc.wait()
            id1.wait()
            id2.wait()

        plsc.subcore_barrier()
        pltpu.sync_copy(acc.at[pl.ds(s * RPS, RPS)],
                        out_hbm.at[c].at[pl.ds(s * RPS, RPS)])

    return spmm_kernel


def _tc_scale_xw(x, w, counts):
    def body(x_ref, w_ref, cnt_ref, o_ref):
        dinv = lax.rsqrt(cnt_ref[0, :, :1] + cnt_ref[1, :, :1] + 1.0)
        xw = jnp.dot(x_ref[...], w_ref[...], preferred_element_type=jnp.float32)
        o_ref[...] = xw * dinv

    return pl.pallas_call(
        body,
        out_shape=jax.ShapeDtypeStruct((x.shape[0], w.shape[1]), jnp.float32),
    )(x, w, counts)


def _tc_combine_next(accl, accr, xs, counts, b, w):
    half = xs.shape[1] // 2

    def body(al_ref, ar_ref, xs_ref, cnt_ref, b_ref, w_ref, o_ref):
        dinv = lax.rsqrt(cnt_ref[0, :, :1] + cnt_ref[1, :, :1] + 1.0)
        left = al_ref[0] + al_ref[1] + xs_ref[:, :half]
        right = ar_ref[0] + ar_ref[1] + xs_ref[:, half:]
        h = dinv * jnp.concatenate([left, right], axis=1) + b_ref[...]
        h = jnp.maximum(h, 0.0)
        hw = jnp.dot(h, w_ref[...], preferred_element_type=jnp.float32)
        o_ref[...] = hw * dinv

    return pl.pallas_call(
        body,
        out_shape=jax.ShapeDtypeStruct((xs.shape[0], w.shape[1]), jnp.float32),
    )(accl, accr, xs, counts, b, w)


def _tc_final(acc, xs, counts, b, n):
    def body(a_ref, xs_ref, cnt_ref, b_ref, o_ref):
        dinv = lax.rsqrt(cnt_ref[0, :n, :1] + cnt_ref[1, :n, :1] + 1.0)
        o_ref[...] = (dinv * (a_ref[0, :n] + a_ref[1, :n] + xs_ref[:n])
                      + b_ref[...])

    return pl.pallas_call(
        body,
        out_shape=jax.ShapeDtypeStruct((n, xs.shape[1]), jnp.float32),
    )(acc, xs, counts, b)


def kernel(x, edge_index, W1, b1, W2, b2):
    n = x.shape[0]
    e = edge_index.shape[1]
    gran = NW * K * 8
    e_pad = ((e + gran - 1) // gran) * gran

    e_rows = e_pad // K + 8
    src = edge_index[0].astype(jnp.int32)
    dst = edge_index[1].astype(jnp.int32)
    pad = jnp.full((e_rows * K - e,), N_NODES, dtype=jnp.int32)
    src = jnp.concatenate([src, pad]).reshape(e_rows, K)
    dst = jnp.concatenate([dst, pad]).reshape(e_rows, K)

    ones_cnt = jnp.ones((K, CNT_W), jnp.float32)
    zeros_cnt = jnp.zeros((N_ACC, CNT_W), jnp.float32)
    zeros_h = jnp.zeros((N_ACC, HID_DIM), jnp.float32)
    zeros_l = jnp.zeros((N_ACC, LAT_DIM), jnp.float32)

    counts = _make_count(e_pad)(dst, ones_cnt, zeros_cnt)

    x_pad = jnp.pad(x, ((0, N_ACC - n), (0, 0)))
    xs1 = _tc_scale_xw(x_pad, W1, counts)
    half = HID_DIM // 2

    split = (92, 68)
    acc1l = _make_spmm_spmem(e_pad, half, 4, split, 0)(xs1, src, dst, zeros_l)
    acc1r = _make_spmm_spmem(e_pad, half, 4, split, half)(
        xs1, src, dst, zeros_l)
    xs2 = _tc_combine_next(acc1l, acc1r, xs1, counts,
                           b1.reshape(1, -1), W2)

    acc2 = _make_spmm_spmem(e_pad, half, 4, split, 0)(xs2, src, dst, zeros_l)
    out = _tc_final(acc2, xs2, counts, b2.reshape(1, -1), n)
    return out


# speedup vs baseline: 2.5549x; 1.0079x over previous
import functools

import jax
import jax.numpy as jnp
from jax import lax
from jax.experimental import pallas as pl
from jax.experimental.pallas import tpu as pltpu
from jax.experimental.pallas import tpu_sc as plsc

N_NODES = 10000
IN_DIM = 128
HID_DIM = 128
LAT_DIM = 64

NC, NS = 2, 16
NW = NC * NS
K = 128
N_ACC = 10112
CNT_W = 16
RPS = N_ACC // NS


def _mesh():
    return plsc.VectorSubcoreMesh(core_axis_name="c", subcore_axis_name="s",
                                  num_cores=NC, num_subcores=NS)


_SC_PARAMS = pltpu.CompilerParams(use_tc_tiling_on_sc=False)


@functools.lru_cache(maxsize=None)
def _make_count(e_pad, nbuf=8):
    n_chunks = e_pad // NW // K

    @functools.partial(
        pl.kernel,
        out_type=jax.ShapeDtypeStruct((NC, N_ACC, CNT_W), jnp.float32),
        mesh=_mesh(),
        compiler_params=_SC_PARAMS,
        scratch_types=(
            [pltpu.VMEM((n_chunks, K), jnp.int32),
             pltpu.VMEM((K, CNT_W), jnp.float32)]
            + [pltpu.SemaphoreType.DMA] * nbuf
            + [pltpu.VMEM_SHARED((N_ACC, CNT_W), jnp.float32)]
        ),
    )
    def count_kernel(dst_hbm, ones_hbm, zeros_hbm, out_hbm, *refs):
        idxb, onesb = refs[0], refs[1]
        sems = refs[2:2 + nbuf]
        acc = refs[2 + nbuf]
        c = lax.axis_index("c")
        s = lax.axis_index("s")
        wid = c * NS + s
        pltpu.sync_copy(ones_hbm, onesb)
        pltpu.sync_copy(dst_hbm.at[pl.ds(wid * n_chunks, n_chunks)], idxb)
        pltpu.sync_copy(zeros_hbm.at[pl.ds(s * RPS, RPS)],
                        acc.at[pl.ds(s * RPS, RPS)])
        plsc.subcore_barrier()

        @pl.loop(0, n_chunks, step=nbuf)
        def _(j):
            descs = [pltpu.async_copy(onesb, acc.at[idxb.at[j + b]],
                                      sems[b], add=True)
                     for b in range(nbuf)]
            for dsc in descs:
                dsc.wait()

        plsc.subcore_barrier()
        pltpu.sync_copy(acc.at[pl.ds(s * RPS, RPS)],
                        out_hbm.at[c].at[pl.ds(s * RPS, RPS)])

    return count_kernel


@functools.lru_cache(maxsize=None)
def _make_spmm_spmem(e_pad, d, nbuf, split=None, col=0):
    n_chunks = e_pad // NW // K
    nc0, nc1 = split if split else (n_chunks, n_chunks)
    assert nc0 % nbuf == 0 and nc1 % nbuf == 0
    assert NS * (nc0 + nc1) == e_pad // K

    @functools.partial(
        pl.kernel,
        out_type=jax.ShapeDtypeStruct((NC, N_ACC, d), jnp.float32),
        mesh=_mesh(),
        compiler_params=_SC_PARAMS,
        scratch_types=(
            [pltpu.VMEM((2, nbuf, K), jnp.int32),
             pltpu.VMEM((2, nbuf, K), jnp.int32)]
            + [pltpu.VMEM((K, d), jnp.float32)] * nbuf
            + [pltpu.SemaphoreType.DMA] * (1 + 2 * nbuf)
            + [pltpu.VMEM_SHARED((N_ACC, d), jnp.float32),
               pltpu.VMEM_SHARED((N_ACC, d), jnp.float32)]
        ),
    )
    def spmm_kernel(xs_hbm, src_hbm, dst_hbm, zeros_hbm, out_hbm, *refs):
        srcb, dstb = refs[0], refs[1]
        rows = refs[2:2 + nbuf]
        isem = refs[2 + nbuf]
        gsems = refs[3 + nbuf:3 + 2 * nbuf]
        ssems = refs[3 + 2 * nbuf:3 + 3 * nbuf]
        xs_sp = refs[3 + 3 * nbuf]
        acc = refs[4 + 3 * nbuf]
        c = lax.axis_index("c")
        s = lax.axis_index("s")
        tbase = jnp.where(c == 0, s * nc0, NS * nc0 + s * nc1)
        my_groups = jnp.where(c == 0, nc0 // nbuf, nc1 // nbuf)
        pltpu.sync_copy(src_hbm.at[pl.ds(tbase, nbuf)], srcb.at[0])
        pltpu.sync_copy(dst_hbm.at[pl.ds(tbase, nbuf)], dstb.at[0])
        pltpu.sync_copy(xs_hbm.at[pl.ds(s * RPS, RPS), pl.ds(col, d)],
                        xs_sp.at[pl.ds(s * RPS, RPS)])
        pltpu.sync_copy(zeros_hbm.at[pl.ds(s * RPS, RPS)],
                        acc.at[pl.ds(s * RPS, RPS)])
        plsc.subcore_barrier()

        @pl.loop(0, my_groups)
        def _(g):
            p = lax.rem(g, 2)
            pn = 1 - p
            nbase = tbase + (g + 1) * nbuf
            id1 = pltpu.async_copy(src_hbm.at[pl.ds(nbase, nbuf)],
                                   srcb.at[pn], isem)
            id2 = pltpu.async_copy(dst_hbm.at[pl.ds(nbase, nbuf)],
                                   dstb.at[pn], isem)
            gds = [pltpu.async_copy(xs_sp.at[srcb.at[p, b]], rows[b],
                                    gsems[b])
                   for b in range(nbuf)]
            sds = []
            for b in range(nbuf):
                gds[b].wait()
                sds.append(pltpu.async_copy(rows[b], acc.at[dstb.at[p, b]],
                                            ssems[b], add=True))
            for dsc in sds:
                dsc.wait()
            id1.wait()
            id2.wait()

        plsc.subcore_barrier()
        pltpu.sync_copy(acc.at[pl.ds(s * RPS, RPS)],
                        out_hbm.at[c].at[pl.ds(s * RPS, RPS)])

    return spmm_kernel


def _tc_scale_xw(x, w, counts):
    def body(x_ref, w_ref, cnt_ref, o_ref):
        dinv = lax.rsqrt(cnt_ref[0, :, :1] + cnt_ref[1, :, :1] + 1.0)
        xw = jnp.dot(x_ref[...], w_ref[...], preferred_element_type=jnp.float32)
        o_ref[...] = xw * dinv

    return pl.pallas_call(
        body,
        out_shape=jax.ShapeDtypeStruct((x.shape[0], w.shape[1]), jnp.float32),
    )(x, w, counts)


def _tc_combine_next(accl, accr, xs, counts, b, w):
    half = xs.shape[1] // 2

    def body(al_ref, ar_ref, xs_ref, cnt_ref, b_ref, w_ref, o_ref):
        dinv = lax.rsqrt(cnt_ref[0, :, :1] + cnt_ref[1, :, :1] + 1.0)
        left = al_ref[0] + al_ref[1] + xs_ref[:, :half]
        right = ar_ref[0] + ar_ref[1] + xs_ref[:, half:]
        h = dinv * jnp.concatenate([left, right], axis=1) + b_ref[...]
        h = jnp.maximum(h, 0.0)
        hw = jnp.dot(h, w_ref[...], preferred_element_type=jnp.float32)
        o_ref[...] = hw * dinv

    return pl.pallas_call(
        body,
        out_shape=jax.ShapeDtypeStruct((xs.shape[0], w.shape[1]), jnp.float32),
    )(accl, accr, xs, counts, b, w)


def _tc_final(acc, xs, counts, b, n):
    def body(a_ref, xs_ref, cnt_ref, b_ref, o_ref):
        dinv = lax.rsqrt(cnt_ref[0, :n, :1] + cnt_ref[1, :n, :1] + 1.0)
        o_ref[...] = (dinv * (a_ref[0, :n] + a_ref[1, :n] + xs_ref[:n])
                      + b_ref[...])

    return pl.pallas_call(
        body,
        out_shape=jax.ShapeDtypeStruct((n, xs.shape[1]), jnp.float32),
    )(acc, xs, counts, b)


def kernel(x, edge_index, W1, b1, W2, b2):
    n = x.shape[0]
    e = edge_index.shape[1]
    gran = NW * K * 8
    e_pad = ((e + gran - 1) // gran) * gran

    e_rows = e_pad // K + 8
    src = edge_index[0].astype(jnp.int32)
    dst = edge_index[1].astype(jnp.int32)
    pad = jnp.full((e_rows * K - e,), N_NODES, dtype=jnp.int32)
    src = jnp.concatenate([src, pad]).reshape(e_rows, K)
    dst = jnp.concatenate([dst, pad]).reshape(e_rows, K)

    ones_cnt = jnp.ones((K, CNT_W), jnp.float32)
    zeros_cnt = jnp.zeros((N_ACC, CNT_W), jnp.float32)
    zeros_l = jnp.zeros((N_ACC, LAT_DIM), jnp.float32)

    counts = _make_count(e_pad)(dst, ones_cnt, zeros_cnt)

    x_pad = jnp.pad(x, ((0, N_ACC - n), (0, 0)))
    xs1 = _tc_scale_xw(x_pad, W1, counts)
    half = HID_DIM // 2

    split = (92, 68)
    acc1l = _make_spmm_spmem(e_pad, half, 4, split, 0)(xs1, src, dst, zeros_l)
    acc1r = _make_spmm_spmem(e_pad, half, 4, split, half)(
        xs1, src, dst, zeros_l)
    xs2 = _tc_combine_next(acc1l, acc1r, xs1, counts,
                           b1.reshape(1, -1), W2)

    acc2 = _make_spmm_spmem(e_pad, half, 4, split, 0)(xs2, src, dst, zeros_l)
    out = _tc_final(acc2, xs2, counts, b2.reshape(1, -1), n)
    return out
